# baseline (speedup 1.0000x reference)
import functools
import math

import jax
import jax.numpy as jnp
from jax import lax
from jax.experimental import pallas as pl
from jax.experimental.pallas import tpu as pltpu
from jax.experimental.pallas import tpu_sc as plsc

_NA = 100
_NF = 512
_NW = 32
_FPT = _NF // _NW
_PADA = 120
_NB = 180
_NBP = 192
_NBLK = 7

_ACOS = (1.5707963050, -0.2145988016, 0.0889789874, -0.0501743046,
         0.0308918810, -0.0170881256, 0.0066700901, -0.0012624911)
_PI = math.pi
_USCALE = 179.0 / math.pi


def _rsqrt_nr(a):
    bits = lax.bitcast_convert_type(a, jnp.int32)
    y = lax.bitcast_convert_type(jnp.int32(0x5F3759DF) - (bits >> 1),
                                 jnp.float32)
    for _ in range(3):
        y = y * (1.5 - 0.5 * a * y * y)
    return y


def _sc_body(x_hbm, out_hbm, xyz_v, hist_v, loc_v):
    wid = lax.axis_index("c") * 16 + lax.axis_index("s")
    pltpu.sync_copy(x_hbm.at[wid], xyz_v)
    lanes = lax.iota(jnp.int32, 16)
    lane_base = lanes * _NBP
    zero16 = jnp.zeros((16,), jnp.float32)
    for lane in range(16):
        for b in range(_NBP // 16):
            hist_v[pl.ds(lane * _NBP + b * 16, 16)] = zero16

    def frame_body(ff, carry):
        for cb in range(_NBLK):
            base = 1 + cb * 16
            dotv = zero16
            n1 = zero16
            n2 = zero16
            ws1 = zero16
            ws2 = zero16
            for d in range(3):
                left = xyz_v[d, ff, pl.ds(base - 1, 16)]
                ctr = xyz_v[d, ff, pl.ds(base, 16)]
                right = xyz_v[d, ff, pl.ds(base + 1, 16)]
                v1 = left - ctr
                v2 = right - ctr
                dotv = dotv + v1 * v2
                n1 = n1 + v1 * v1
                n2 = n2 + v2 * v2
                w1 = v1 + (jnp.where(v1 >= 10.0, -20.0, 0.0)
                           + jnp.where(v1 < -10.0, 20.0, 0.0))
                w2 = v2 + (jnp.where(v2 >= 10.0, -20.0, 0.0)
                           + jnp.where(v2 < -10.0, 20.0, 0.0))
                ws1 = ws1 + w1 * w1
                ws2 = ws2 + w2 * w2
            valid = ((ws1 < 9.0) & (ws1 != 0.0)
                     & (ws2 < 9.0) & (ws2 != 0.0))
            if cb == _NBLK - 1:
                valid = valid & (lanes < (_NA - 2) - (base - 1))
            cosv = dotv * _rsqrt_nr(n1 * n2)
            cosv = jnp.clip(cosv, -1.0 + 1e-7, 1.0 - 1e-7)
            cosv = jnp.where(valid, cosv, 0.0)
            t = jnp.abs(cosv)
            p = jnp.full((16,), _ACOS[7], jnp.float32)
            for c in (_ACOS[6], _ACOS[5], _ACOS[4], _ACOS[3],
                      _ACOS[2], _ACOS[1], _ACOS[0]):
                p = p * t + c
            omt = 1.0 - t
            root = omt * _rsqrt_nr(omt)
            th = root * p
            theta = jnp.where(cosv < 0.0, _PI - th, th)
            u = theta * _USCALE
            s_i = jnp.clip(u.astype(jnp.int32) - 7, 0, _NB - 16)
            s_f = s_i.astype(jnp.float32)
            for j in range(16):
                dd = u - (s_f + float(j))
                w = jnp.exp(-0.5 * dd * dd)
                plsc.addupdate_scatter(hist_v, [lane_base + s_i + j], w,
                                       mask=valid)
        return carry

    lax.fori_loop(0, _FPT, frame_body, 0)

    for b in range(_NBP // 16):
        acc = hist_v[pl.ds(b * 16, 16)]
        for lane in range(1, 16):
            acc = acc + hist_v[pl.ds(lane * _NBP + b * 16, 16)]
        loc_v[pl.ds(b * 16, 16)] = acc
    pltpu.sync_copy(loc_v, out_hbm.at[wid])


_sc_hist = pl.kernel(
    _sc_body,
    out_type=jax.ShapeDtypeStruct((_NW, _NBP), jnp.float32),
    mesh=plsc.VectorSubcoreMesh(core_axis_name="c", subcore_axis_name="s",
                                num_cores=2, num_subcores=16),
    compiler_params=pltpu.CompilerParams(needs_layout_passes=False),
    scratch_types=[
        pltpu.VMEM((3, _FPT, _PADA), jnp.float32),
        pltpu.VMEM((16 * _NBP,), jnp.float32),
        pltpu.VMEM((_NBP,), jnp.float32),
    ],
)


def kernel(xyz):
    xyz = xyz.reshape(-1, _NA, 3)
    x = jnp.transpose(xyz, (2, 0, 1))
    x = jnp.pad(x, ((0, 0), (0, 0), (0, _PADA - _NA)))
    x = x.reshape(3, _NW, _FPT, _PADA).transpose(1, 0, 2, 3)
    parts = _sc_hist(x)
    hist = parts.sum(0)[:_NB]

    offset = jnp.linspace(0.0, 180.0, _NB)
    width = offset[1] - offset[0]
    coeff = -0.5 / width ** 2
    fb_v1 = xyz[0, 1] - xyz[0, 0]
    fb_v2 = xyz[0, 2] - xyz[0, 0]
    fb_dot = (fb_v1 * fb_v2).sum()
    fb_nrm = jnp.sqrt((fb_v1 ** 2).sum() * (fb_v2 ** 2).sum())
    fb_cos = jnp.clip(fb_dot / fb_nrm, -1.0 + 1e-7, 1.0 - 1e-7)
    fb_ang = jnp.arccos(fb_cos) * 180.0 / _PI
    fb_hist = jnp.exp(coeff * (fb_ang - offset) ** 2)

    count = jnp.where(hist.sum() > 0.0, hist, fb_hist)
    return count / count.sum()
